# baseline (speedup 1.0000x reference)
import functools

import jax
import jax.numpy as jnp
import numpy as np
from jax import lax
from jax.experimental import pallas as pl
from jax.experimental.pallas import tpu as pltpu
from jax.experimental.pallas import tpu_sc as plsc
from jax._src.pallas import mpmd as _mpmd

_P = 1.0
_LB = 0.8
_HB = 1.2
_F = 4096
_N_TS = 65536
_D = 1024

with jax.default_device(jax.local_devices(backend="cpu")[0]):
    _key = jax.random.key(42)
    _k1, _k2, _k3 = jax.random.split(_key, 3)
    _coin = float(jax.random.uniform(_k1, ()))
    _selection = np.asarray(jax.random.choice(_k2, _N_TS, (_F,), replace=False))
    _factor = float((_HB - _LB) * jax.random.uniform(_k3, ()) + _LB)
    _apply = bool(_coin < _P)

_NC = 2
_NS = 16
_NW = _NC * _NS
_RPW = _N_TS // _NW
_CH = 32
_LANES = 16

_NCH = _F // (_NW * _CH)
_owned = _selection.astype(np.int32).reshape(_NW, _NCH, _CH)

_NDMA = 16
_STRIPE = _N_TS // _NDMA


def _tc_copy_body(x_hbm, o_hbm, sem):
    for i in range(_NDMA):
        sl = pl.ds(i * _STRIPE, _STRIPE)
        pltpu.async_copy(x_hbm.at[sl], o_hbm.at[sl], sem)
    for i in range(_NDMA):
        sl = pl.ds(i * _STRIPE, _STRIPE)
        pltpu.make_async_copy(x_hbm.at[sl], o_hbm.at[sl], sem).wait()


def _tc_copy(data):
    return pl.pallas_call(
        _tc_copy_body,
        in_specs=[pl.BlockSpec(memory_space=pltpu.HBM)],
        out_specs=pl.BlockSpec(memory_space=pltpu.HBM),
        scratch_shapes=[pltpu.SemaphoreType.DMA],
        out_shape=jax.ShapeDtypeStruct((_N_TS, _D), jnp.float32),
    )(data)


_mesh = plsc.VectorSubcoreMesh(core_axis_name="c", subcore_axis_name="s")


def _sc_scatter_body(copied_hbm, data_hbm, owned_hbm, out_hbm, idx_v, buf0,
                     buf1, g0, g1, s0, s1):
    del copied_hbm
    wid = lax.axis_index("s") * _NC + lax.axis_index("c")
    bufs = (buf0, buf1)
    gsems = (g0, g1)
    ssems = (s0, s1)

    def _gather(c, b):
        pltpu.async_copy(data_hbm.at[idx_v.at[c]], bufs[b], gsems[b])

    def _gather_wait(c, b):
        pltpu.make_async_copy(data_hbm.at[idx_v.at[c]], bufs[b],
                              gsems[b]).wait()

    def _scatter(c, b):
        pltpu.async_copy(bufs[b], out_hbm.at[idx_v.at[c]], ssems[b])

    def _scatter_wait(c, b):
        pltpu.make_async_copy(bufs[b], out_hbm.at[idx_v.at[c]],
                              ssems[b]).wait()

    pltpu.sync_copy(owned_hbm.at[wid], idx_v)
    _gather(0, 0)
    for c in range(_NCH):
        b = c % 2
        _gather_wait(c, b)
        if c + 1 < _NCH:
            if c >= 1:
                _scatter_wait(c - 1, 1 - b)
            _gather(c + 1, 1 - b)

        def _scale_row(r, carry, _buf=bufs[b]):
            for j in range(_D // _LANES):
                sl = pl.ds(j * _LANES, _LANES)
                _buf[r, sl] = _buf[r, sl] * _factor
            return carry

        lax.fori_loop(0, _CH, _scale_row, 0)
        _scatter(c, b)
    _scatter_wait(_NCH - 2, _NCH % 2)
    _scatter_wait(_NCH - 1, (_NCH - 1) % 2)


_sc_scatter = _mpmd._mpmd_map(
    [(_mesh, _sc_scatter_body)],
    jax.ShapeDtypeStruct((_N_TS, _D), jnp.float32),
    input_output_aliases={0: 0},
    scratch_types=[
        pltpu.VMEM((_NCH, _CH), jnp.int32),
        pltpu.VMEM((_CH, _D), jnp.float32),
        pltpu.VMEM((_CH, _D), jnp.float32),
        pltpu.SemaphoreType.DMA,
        pltpu.SemaphoreType.DMA,
        pltpu.SemaphoreType.DMA,
        pltpu.SemaphoreType.DMA,
    ],
)


def kernel(data):
    if not _apply:
        return data
    copied = _tc_copy(data)
    return _sc_scatter(copied, data, jnp.asarray(_owned))


# speedup vs baseline: 39.9625x; 39.9625x over previous
import functools

import jax
import jax.numpy as jnp
import numpy as np
from jax import lax
from jax.experimental import pallas as pl
from jax.experimental.pallas import tpu as pltpu
from jax.experimental.pallas import tpu_sc as plsc
from jax._src.pallas import mpmd as _mpmd

_P = 1.0
_LB = 0.8
_HB = 1.2
_F = 4096
_N_TS = 65536
_D = 1024

with jax.default_device(jax.local_devices(backend="cpu")[0]):
    _key = jax.random.key(42)
    _k1, _k2, _k3 = jax.random.split(_key, 3)
    _coin = float(jax.random.uniform(_k1, ()))
    _selection = np.asarray(jax.random.choice(_k2, _N_TS, (_F,), replace=False))
    _factor = float((_HB - _LB) * jax.random.uniform(_k3, ()) + _LB)
    _apply = bool(_coin < _P)

_NC = 2
_NS = 16
_NW = _NC * _NS
_RPW = _N_TS // _NW
_CH = 32
_LANES = 16

_NCH = _F // (_NW * _CH)
_owned = _selection.astype(np.int32).reshape(_NW, _NCH, _CH)

_ROWS = 2048


def _tc_copy_body(x_ref, o_ref):
    o_ref[...] = x_ref[...]


def _tc_copy(data):
    return pl.pallas_call(
        _tc_copy_body,
        grid=(_N_TS // _ROWS,),
        in_specs=[pl.BlockSpec((_ROWS, _D), lambda i: (i, 0))],
        out_specs=pl.BlockSpec((_ROWS, _D), lambda i: (i, 0)),
        out_shape=jax.ShapeDtypeStruct((_N_TS, _D), jnp.float32),
    )(data)


_mesh = plsc.VectorSubcoreMesh(core_axis_name="c", subcore_axis_name="s")


def _sc_scatter_body(copied_hbm, data_hbm, owned_hbm, out_hbm, idx_v, buf0,
                     buf1, g0, g1, s0, s1):
    del copied_hbm
    wid = lax.axis_index("s") * _NC + lax.axis_index("c")
    bufs = (buf0, buf1)
    gsems = (g0, g1)
    ssems = (s0, s1)

    def _gather(c, b):
        pltpu.async_copy(data_hbm.at[idx_v.at[c]], bufs[b], gsems[b])

    def _gather_wait(c, b):
        pltpu.make_async_copy(data_hbm.at[idx_v.at[c]], bufs[b],
                              gsems[b]).wait()

    def _scatter(c, b):
        pltpu.async_copy(bufs[b], out_hbm.at[idx_v.at[c]], ssems[b])

    def _scatter_wait(c, b):
        pltpu.make_async_copy(bufs[b], out_hbm.at[idx_v.at[c]],
                              ssems[b]).wait()

    pltpu.sync_copy(owned_hbm.at[wid], idx_v)
    _gather(0, 0)
    for c in range(_NCH):
        b = c % 2
        _gather_wait(c, b)
        if c + 1 < _NCH:
            if c >= 1:
                _scatter_wait(c - 1, 1 - b)
            _gather(c + 1, 1 - b)

        def _scale_rows(r2, carry, _buf=bufs[b]):
            for u in range(2):
                r = 2 * r2 + u
                for j in range(_D // _LANES):
                    sl = pl.ds(j * _LANES, _LANES)
                    _buf[r, sl] = _buf[r, sl] * _factor
            return carry

        lax.fori_loop(0, _CH // 2, _scale_rows, 0)
        _scatter(c, b)
    _scatter_wait(_NCH - 2, _NCH % 2)
    _scatter_wait(_NCH - 1, (_NCH - 1) % 2)


_sc_scatter = _mpmd._mpmd_map(
    [(_mesh, _sc_scatter_body)],
    jax.ShapeDtypeStruct((_N_TS, _D), jnp.float32),
    input_output_aliases={0: 0},
    scratch_types=[
        pltpu.VMEM((_NCH, _CH), jnp.int32),
        pltpu.VMEM((_CH, _D), jnp.float32),
        pltpu.VMEM((_CH, _D), jnp.float32),
        pltpu.SemaphoreType.DMA,
        pltpu.SemaphoreType.DMA,
        pltpu.SemaphoreType.DMA,
        pltpu.SemaphoreType.DMA,
    ],
)


def kernel(data):
    if not _apply:
        return data
    copied = _tc_copy(data)
    return _sc_scatter(copied, data, jnp.asarray(_owned))


# speedup vs baseline: 40.6358x; 1.0168x over previous
import functools

import jax
import jax.numpy as jnp
import numpy as np
from jax import lax
from jax.experimental import pallas as pl
from jax.experimental.pallas import tpu as pltpu
from jax.experimental.pallas import tpu_sc as plsc
from jax._src.pallas import mpmd as _mpmd

_P = 1.0
_LB = 0.8
_HB = 1.2
_F = 4096
_N_TS = 65536
_D = 1024

with jax.default_device(jax.local_devices(backend="cpu")[0]):
    _key = jax.random.key(42)
    _k1, _k2, _k3 = jax.random.split(_key, 3)
    _coin = float(jax.random.uniform(_k1, ()))
    _selection = np.asarray(jax.random.choice(_k2, _N_TS, (_F,), replace=False))
    _factor = float((_HB - _LB) * jax.random.uniform(_k3, ()) + _LB)
    _apply = bool(_coin < _P)

_NC = 2
_NS = 16
_NW = _NC * _NS
_RPW = _N_TS // _NW
_CH = 32
_LANES = 16

_NCH = _F // (_NW * _CH)
_owned = _selection.astype(np.int32).reshape(_NW, _NCH, _CH)

_ROWS = 2048


def _tc_copy_body(x_ref, o_ref):
    o_ref[...] = x_ref[...]


def _tc_copy(data):
    return pl.pallas_call(
        _tc_copy_body,
        grid=(_N_TS // _ROWS,),
        in_specs=[pl.BlockSpec((_ROWS, _D), lambda i: (i, 0))],
        out_specs=pl.BlockSpec((_ROWS, _D), lambda i: (i, 0)),
        out_shape=jax.ShapeDtypeStruct((_N_TS, _D), jnp.float32),
    )(data)


_mesh = plsc.VectorSubcoreMesh(core_axis_name="c", subcore_axis_name="s")


def _sc_scatter_body(copied_hbm, data_hbm, owned_hbm, out_hbm, idxs, buf0,
                     buf1, g0, g1, s0, s1):
    del copied_hbm
    wid = lax.axis_index("s") * _NC + lax.axis_index("c")
    bufs = (buf0, buf1)
    gsems = (g0, g1)
    ssems = (s0, s1)

    def _gather(c, b):
        pltpu.async_copy(data_hbm.at[idxs[c]], bufs[b], gsems[b])

    def _gather_wait(c, b):
        pltpu.make_async_copy(data_hbm.at[idxs[c]], bufs[b],
                              gsems[b]).wait()

    def _scatter(c, b):
        pltpu.async_copy(bufs[b], out_hbm.at[idxs[c]], ssems[b])

    def _scatter_wait(c, b):
        pltpu.make_async_copy(bufs[b], out_hbm.at[idxs[c]],
                              ssems[b]).wait()

    for c in range(_NCH):
        pltpu.sync_copy(owned_hbm.at[wid, c], idxs[c])
    _gather(0, 0)
    for c in range(_NCH):
        b = c % 2
        _gather_wait(c, b)
        if c + 1 < _NCH:
            if c >= 1:
                _scatter_wait(c - 1, 1 - b)
            _gather(c + 1, 1 - b)

        def _scale_row(r, carry, _buf=bufs[b]):
            for j in range(_D // _LANES):
                sl = pl.ds(j * _LANES, _LANES)
                _buf[r, sl] = _buf[r, sl] * _factor
            return carry

        lax.fori_loop(0, _CH, _scale_row, 0)
        _scatter(c, b)
    _scatter_wait(_NCH - 2, _NCH % 2)
    _scatter_wait(_NCH - 1, (_NCH - 1) % 2)


_sc_scatter = _mpmd._mpmd_map(
    [(_mesh, _sc_scatter_body)],
    jax.ShapeDtypeStruct((_N_TS, _D), jnp.float32),
    input_output_aliases={0: 0},
    scratch_types=[
        [pltpu.VMEM((_CH,), jnp.int32) for _ in range(_NCH)],
        pltpu.VMEM((_CH, _D), jnp.float32),
        pltpu.VMEM((_CH, _D), jnp.float32),
        pltpu.SemaphoreType.DMA,
        pltpu.SemaphoreType.DMA,
        pltpu.SemaphoreType.DMA,
        pltpu.SemaphoreType.DMA,
    ],
)


def kernel(data):
    if not _apply:
        return data
    copied = _tc_copy(data)
    return _sc_scatter(copied, data, jnp.asarray(_owned))
